# baseline (speedup 1.0000x reference)
import functools

import jax
import jax.numpy as jnp
from jax import lax
from jax.experimental import pallas as pl
from jax.experimental.pallas import tpu as pltpu
from jax.experimental.pallas import tpu_sc as plsc

VOCAB = 1000000
D_MODEL = 64
BATCH = 16384
SEQ = 50
NUM_CORES = 2
NUM_SUBCORES = 16
NW = NUM_CORES * NUM_SUBCORES
CHUNK = 128
NBBLK = BATCH // CHUNK
BBLK_PER_W = NBBLK // NW
N_CHUNK = SEQ * BBLK_PER_W
DBLK = D_MODEL // 8

_mesh = plsc.VectorSubcoreMesh(core_axis_name="c", subcore_axis_name="s")


@functools.partial(
    pl.kernel,
    mesh=_mesh,
    out_type=jax.ShapeDtypeStruct((SEQ, DBLK, NBBLK, 8, CHUNK), jnp.float32),
    scratch_types=[
        pltpu.VMEM((SEQ, BBLK_PER_W * CHUNK), jnp.int32),
        pltpu.VMEM((2, CHUNK), jnp.int32),
        pltpu.VMEM((CHUNK, 2 * D_MODEL), jnp.float32),
        pltpu.VMEM((CHUNK, 2 * D_MODEL), jnp.float32),
        pltpu.VMEM((D_MODEL, CHUNK + 1), jnp.float32),
        pltpu.VMEM((D_MODEL, CHUNK + 1), jnp.float32),
        pltpu.SemaphoreType.DMA,
        pltpu.SemaphoreType.DMA,
        pltpu.SemaphoreType.DMA,
        pltpu.SemaphoreType.DMA,
    ],
    compiler_params=pltpu.CompilerParams(
        use_tc_tiling_on_sc=True, needs_layout_passes=False
    ),
)
def _gather_kernel(idx_hbm, table_hbm, out_hbm, idx_v, hidx, g0, g1, t0, t1,
                   gsem0, gsem1, osem0, osem1):
    wid = lax.axis_index("s") * NUM_CORES + lax.axis_index("c")
    gbufs = (g0, g1)
    tbufs = (t0, t1)
    gsems = (gsem0, gsem1)
    osems = (osem0, osem1)

    pltpu.sync_copy(
        idx_hbm.at[:, pl.ds(wid * BBLK_PER_W * CHUNK, BBLK_PER_W * CHUNK)],
        idx_v,
    )

    def chunk_sj(c):
        return c // BBLK_PER_W, c % BBLK_PER_W

    def compute_hidx(c, b):
        s, j = chunk_sj(c)
        for k in range(8):
            v16 = idx_v[s, pl.ds(j * CHUNK + k * 16, 16)]
            hidx[b, pl.ds(k * 16, 16)] = lax.shift_right_logical(v16, 1)

    def gather_descr(b):
        return pltpu.make_async_copy(
            table_hbm.at[hidx.at[b]],
            gbufs[b],
            gsems[b],
        )

    def out_descrs(c, b):
        s, j = chunk_sj(c)
        return [
            pltpu.make_async_copy(
                tbufs[b].at[pl.ds(dblk * 8, 8), pl.ds(0, CHUNK)],
                out_hbm.at[s, dblk, wid * BBLK_PER_W + j],
                osems[b],
            )
            for dblk in range(DBLK)
        ]

    iv16 = lax.iota(jnp.int32, 16)
    ivt = [iv16 + 16 * k for k in range(8)]
    iv_zero = iv16 * 0

    ivd = [iv16 + 16 * k for k in range(4)]

    def transpose_chunk(c, b):
        s, j = chunk_sj(c)
        gbuf, tbuf = gbufs[b], tbufs[b]

        def gbody(g, carry):
            pv = idx_v[s, pl.ds(j * CHUNK + g * 16, 16)]
            offv = (pv & 1) * D_MODEL
            base = g * 16
            for i in range(16):
                t = base + i
                off = offv[i]
                iv_t = iv_zero + t
                vs = [gbuf[t, pl.ds(off + k * 16, 16)] for k in range(4)]
                for k in range(4):
                    plsc.store_scatter(tbuf, [ivd[k], iv_t], vs[k])
            return carry

        lax.fori_loop(0, 8, gbody, 0)

    compute_hidx(0, 0)
    gather_descr(0).start()

    def outer(p, carry):
        for b in range(2):
            c = p * 2 + b
            if b == 0:
                compute_hidx(c + 1, 1)
                gather_descr(1).start()
            else:
                @pl.when(p < N_CHUNK // 2 - 1)
                def _():
                    compute_hidx(c + 1, 0)
                    gather_descr(0).start()
            gather_descr(b).wait()

            @pl.when(p >= 1)
            def _():
                for d in out_descrs(c - 2, b):
                    d.wait()

            transpose_chunk(c, b)
            for d in out_descrs(c, b):
                d.start()
        return carry

    lax.fori_loop(0, N_CHUNK // 2, outer, 0)
    for d in out_descrs(N_CHUNK - 2, 0):
        d.wait()
    for d in out_descrs(N_CHUNK - 1, 1):
        d.wait()


def kernel(token_ids, weights):
    idx_t = token_ids.T
    w2 = weights.reshape(VOCAB // 2, 2 * D_MODEL)
    p5 = _gather_kernel(idx_t, w2)
    return p5.transpose(2, 4, 0, 1, 3).reshape(BATCH, SEQ, D_MODEL)


# speedup vs baseline: 1.8684x; 1.8684x over previous
import functools

import jax
import jax.numpy as jnp
from jax import lax
from jax.experimental import pallas as pl
from jax.experimental.pallas import tpu as pltpu
from jax.experimental.pallas import tpu_sc as plsc

VOCAB = 1000000
D_MODEL = 64
BATCH = 16384
SEQ = 50
NUM_CORES = 2
NUM_SUBCORES = 16
NW = NUM_CORES * NUM_SUBCORES
CHUNK = 128
NBBLK = BATCH // CHUNK
BBLK_PER_W = NBBLK // NW
N_CHUNK = SEQ * BBLK_PER_W
DBLK = D_MODEL // 8

_mesh = plsc.VectorSubcoreMesh(core_axis_name="c", subcore_axis_name="s")


@functools.partial(
    pl.kernel,
    mesh=_mesh,
    out_type=jax.ShapeDtypeStruct((SEQ, DBLK, NBBLK, 8, CHUNK), jnp.float32),
    scratch_types=[
        pltpu.VMEM((SEQ, BBLK_PER_W * CHUNK), jnp.int32),
        pltpu.VMEM((CHUNK, D_MODEL), jnp.float32),
        pltpu.VMEM((CHUNK, D_MODEL), jnp.float32),
        pltpu.VMEM((D_MODEL, CHUNK + 1), jnp.float32),
        pltpu.VMEM((D_MODEL, CHUNK + 1), jnp.float32),
        pltpu.SemaphoreType.DMA,
        pltpu.SemaphoreType.DMA,
        pltpu.SemaphoreType.DMA,
        pltpu.SemaphoreType.DMA,
    ],
    compiler_params=pltpu.CompilerParams(
        use_tc_tiling_on_sc=False, needs_layout_passes=False
    ),
)
def _gather_kernel(idx_hbm, table_hbm, out_hbm, idx_v, g0, g1, t0, t1,
                   gsem0, gsem1, osem0, osem1):
    wid = lax.axis_index("s") * NUM_CORES + lax.axis_index("c")
    gbufs = (g0, g1)
    tbufs = (t0, t1)
    gsems = (gsem0, gsem1)
    osems = (osem0, osem1)

    pltpu.sync_copy(
        idx_hbm.at[:, pl.ds(wid * BBLK_PER_W * CHUNK, BBLK_PER_W * CHUNK)],
        idx_v,
    )

    def chunk_sj(c):
        return c // BBLK_PER_W, c % BBLK_PER_W

    def gather_descr(c, b):
        s, j = chunk_sj(c)
        return pltpu.make_async_copy(
            table_hbm.at[idx_v.at[s, pl.ds(j * CHUNK, CHUNK)]],
            gbufs[b],
            gsems[b],
        )

    def out_descrs(c, b):
        s, j = chunk_sj(c)
        return [
            pltpu.make_async_copy(
                tbufs[b].at[pl.ds(dblk * 8, 8), pl.ds(0, CHUNK)],
                out_hbm.at[s, dblk, wid * BBLK_PER_W + j],
                osems[b],
            )
            for dblk in range(DBLK)
        ]

    iv16 = lax.iota(jnp.int32, 16)
    ivt = [iv16 + 16 * k for k in range(8)]
    iv_zero = iv16 * 0

    ivd = [iv16 + 16 * k for k in range(4)]

    def transpose_chunk(c, b):
        gbuf, tbuf = gbufs[b], tbufs[b]

        def tbody(t, carry):
            iv_t = iv_zero + t
            vs = [gbuf[t, pl.ds(k * 16, 16)] for k in range(4)]
            for k in range(4):
                plsc.store_scatter(tbuf, [ivd[k], iv_t], vs[k])
            return carry

        lax.fori_loop(0, CHUNK, tbody, 0)

    gather_descr(0, 0).start()

    def outer(p, carry):
        for b in range(2):
            c = p * 2 + b
            if b == 0:
                gather_descr(c + 1, 1).start()
            else:
                @pl.when(p < N_CHUNK // 2 - 1)
                def _():
                    gather_descr(c + 1, 0).start()
            gather_descr(c, b).wait()

            @pl.when(p >= 1)
            def _():
                for d in out_descrs(c - 2, b):
                    d.wait()

            transpose_chunk(c, b)
            for d in out_descrs(c, b):
                d.start()
        return carry

    lax.fori_loop(0, N_CHUNK // 2, outer, 0)
    for d in out_descrs(N_CHUNK - 2, 0):
        d.wait()
    for d in out_descrs(N_CHUNK - 1, 1):
        d.wait()


def kernel(token_ids, weights):
    idx_t = token_ids.T
    p5 = _gather_kernel(idx_t, weights)
    return p5.transpose(2, 4, 0, 1, 3).reshape(BATCH, SEQ, D_MODEL)
